# speedup vs baseline: 3.4661x; 3.4661x over previous
import functools

import jax
import jax.numpy as jnp
from jax import lax
from jax.experimental import pallas as pl
from jax.experimental.pallas import tpu as pltpu
from jax.experimental.pallas import tpu_sc as plsc

NBUF = 4
CHUNK = 128


@functools.cache
def _embed_call(N, V, D, NC, NS):
    NW = NC * NS
    per_w = N // NW
    n_chunks = per_w // CHUNK
    n_groups = n_chunks // NBUF
    assert per_w * NW == N and n_chunks * CHUNK == per_w
    assert n_groups * NBUF == n_chunks

    mesh = plsc.VectorSubcoreMesh(core_axis_name="c", subcore_axis_name="s")

    @functools.partial(
        pl.kernel,
        mesh=mesh,
        out_type=jax.ShapeDtypeStruct((N, D), jnp.float32),
        scratch_types=(
            [
                pltpu.VMEM((n_chunks, CHUNK), jnp.int32),
                pltpu.VMEM((NBUF, CHUNK, D), jnp.float32),
            ]
            + [pltpu.SemaphoreType.DMA] * (2 * NBUF)
        ),
    )
    def emb(idx_hbm, table_hbm, out_hbm, idx_v, rows_v, *sems):
        gsem = sems[:NBUF]
        ssem = sems[NBUF:]
        wid = lax.axis_index("s") * NC + lax.axis_index("c")
        base = wid * per_w

        pltpu.sync_copy(idx_hbm.at[wid], idx_v)

        for b in range(NBUF):
            pltpu.async_copy(table_hbm.at[idx_v.at[b]], rows_v.at[b], gsem[b])

        def group(g, carry):
            for b in range(NBUF):
                c = g * NBUF + b
                bp = (b - 1) % NBUF

                @pl.when(c >= 1)
                def _refill():
                    pltpu.make_async_copy(
                        rows_v.at[bp], out_hbm.at[pl.ds(0, CHUNK)], ssem[bp]
                    ).wait()

                    @pl.when(c - 1 + NBUF < n_chunks)
                    def _launch():
                        pltpu.async_copy(
                            table_hbm.at[idx_v.at[c - 1 + NBUF]],
                            rows_v.at[bp],
                            gsem[bp],
                        )

                pltpu.make_async_copy(
                    out_hbm.at[pl.ds(0, CHUNK)], rows_v.at[b], gsem[b]
                ).wait()
                pltpu.async_copy(
                    rows_v.at[b],
                    out_hbm.at[pl.ds(base + c * CHUNK, CHUNK)],
                    ssem[b],
                )
            return carry

        lax.fori_loop(0, n_groups, group, 0)

        pltpu.make_async_copy(
            rows_v.at[NBUF - 1], out_hbm.at[pl.ds(0, CHUNK)], ssem[NBUF - 1]
        ).wait()

    return emb


def kernel(token_ids, weights):
    B, T = token_ids.shape
    V, D = weights.shape
    N = B * T
    info = plsc.get_sparse_core_info()
    NC, NS = info.num_cores, info.num_subcores
    NW = NC * NS
    per_w = N // NW
    n_chunks = per_w // CHUNK
    idx = token_ids.reshape(NW, n_chunks, CHUNK).astype(jnp.int32)
    out = _embed_call(N, V, D, NC, NS)(idx, weights)
    return out.reshape(B, T, D)


# speedup vs baseline: 3.4776x; 1.0033x over previous
import functools

import jax
import jax.numpy as jnp
from jax import lax
from jax.experimental import pallas as pl
from jax.experimental.pallas import tpu as pltpu
from jax.experimental.pallas import tpu_sc as plsc

NBUF = 5
CHUNK = 128


@functools.cache
def _embed_call(N, V, D, NC, NS):
    NW = NC * NS
    per_w = N // NW
    n_chunks = per_w // CHUNK
    n_groups = n_chunks // NBUF
    assert per_w * NW == N and n_chunks * CHUNK == per_w
    assert n_groups * NBUF == n_chunks

    mesh = plsc.VectorSubcoreMesh(core_axis_name="c", subcore_axis_name="s")

    @functools.partial(
        pl.kernel,
        mesh=mesh,
        out_type=jax.ShapeDtypeStruct((N, D), jnp.float32),
        scratch_types=(
            [
                pltpu.VMEM((n_chunks, CHUNK), jnp.int32),
                pltpu.VMEM((NBUF, CHUNK, D), jnp.float32),
            ]
            + [pltpu.SemaphoreType.DMA] * (2 * NBUF)
        ),
    )
    def emb(idx_hbm, table_hbm, out_hbm, idx_v, rows_v, *sems):
        gsem = sems[:NBUF]
        ssem = sems[NBUF:]
        wid = lax.axis_index("s") * NC + lax.axis_index("c")
        base = wid * per_w

        pltpu.sync_copy(idx_hbm.at[wid], idx_v)

        for b in range(NBUF):
            pltpu.async_copy(table_hbm.at[idx_v.at[b]], rows_v.at[b], gsem[b])

        def group(g, carry):
            for b in range(NBUF):
                c = g * NBUF + b
                bp = (b - 1) % NBUF

                @pl.when(c >= 1)
                def _refill():
                    pltpu.make_async_copy(
                        rows_v.at[bp], out_hbm.at[pl.ds(0, CHUNK)], ssem[bp]
                    ).wait()

                    @pl.when(c - 1 + NBUF < n_chunks)
                    def _launch():
                        pltpu.async_copy(
                            table_hbm.at[idx_v.at[c - 1 + NBUF]],
                            rows_v.at[bp],
                            gsem[bp],
                        )

                pltpu.make_async_copy(
                    out_hbm.at[pl.ds(0, CHUNK)], rows_v.at[b], gsem[b]
                ).wait()
                pltpu.async_copy(
                    rows_v.at[b],
                    out_hbm.at[pl.ds(base + c * CHUNK, CHUNK)],
                    ssem[b],
                )
            return carry

        lax.fori_loop(0, n_groups, group, 0)

        pltpu.make_async_copy(
            rows_v.at[NBUF - 1], out_hbm.at[pl.ds(0, CHUNK)], ssem[NBUF - 1]
        ).wait()

    return emb


def kernel(token_ids, weights):
    B, T = token_ids.shape
    V, D = weights.shape
    N = B * T
    info = plsc.get_sparse_core_info()
    NC, NS = info.num_cores, info.num_subcores
    NW = NC * NS
    per_w = N // NW
    n_chunks = per_w // CHUNK
    idx = token_ids.reshape(NW, n_chunks, CHUNK).astype(jnp.int32)
    out = _embed_call(N, V, D, NC, NS)(idx, weights)
    return out.reshape(B, T, D)
